# baseline (speedup 1.0000x reference)
import functools
import math

import jax
import jax.numpy as jnp
from jax import lax
from jax.experimental import pallas as pl
from jax.experimental.pallas import tpu as pltpu
from jax.experimental.pallas import tpu_sc as plsc

_VOCAB = 1000000
_DIM = 64
_B = 4096
_S = 200
_N = _B * _S
_NW = 32
_PER_W = _N // _NW
_C = 128
_NCHUNK = _PER_W // _C
_SCALE = 1.0 / math.sqrt(_DIM)
_LANES = 16


def _embed_kernel(ids_hbm, table_hbm, out_hbm, idx_v, gbuf, sbuf, sem_g, sem_s):
    wid = lax.axis_index("s") * 2 + lax.axis_index("c")
    base = wid * _PER_W

    pltpu.sync_copy(ids_hbm.at[wid], idx_v)

    def chunk(j):
        pltpu.async_copy(table_hbm.at[idx_v.at[j]], gbuf, sem_g).wait()
        def scale_row(i, _):
            for c in range(_DIM // _LANES):
                sl = pl.ds(c * _LANES, _LANES)
                sbuf[i, sl] = gbuf[i, sl] * _SCALE
            return 0
        lax.fori_loop(0, _C, scale_row, 0, unroll=4)
        pltpu.async_copy(sbuf, out_hbm.at[pl.ds(base + j * _C, _C)], sem_s).wait()

    def loop_body(j, _):
        chunk(j)
        return 0

    lax.fori_loop(0, _NCHUNK, loop_body, 0)


@jax.jit
def _embed(ids_grouped, W):
    mesh = plsc.VectorSubcoreMesh(core_axis_name="c", subcore_axis_name="s")
    flat = pl.kernel(
        _embed_kernel,
        mesh=mesh,
        out_type=jax.ShapeDtypeStruct((_N, _DIM), jnp.float32),
        scratch_types=[
            pltpu.VMEM((_NCHUNK, _C), jnp.int32),
            pltpu.VMEM((_C, _DIM), jnp.float32),
            pltpu.VMEM((_C, _DIM), jnp.float32),
            pltpu.SemaphoreType.DMA,
            pltpu.SemaphoreType.DMA,
        ],
        compiler_params=pltpu.CompilerParams(use_tc_tiling_on_sc=False),
    )(ids_grouped, W)
    return flat


def kernel(ids, W):
    ids_grouped = ids.astype(jnp.int32).reshape(_NW, _NCHUNK, _C)
    flat = _embed(ids_grouped, W)
    return flat.reshape(_B, _S, _DIM)


# speedup vs baseline: 1.1785x; 1.1785x over previous
import math

import jax
import jax.numpy as jnp
from jax import lax
from jax.experimental import pallas as pl
from jax.experimental.pallas import tpu as pltpu
from jax.experimental.pallas import tpu_sc as plsc

_VOCAB = 1000000
_DIM = 64
_B = 4096
_S = 200
_N = _B * _S
_NW = 32
_PER_W = _N // _NW
_C = 128
_NCHUNK = _PER_W // _C
_SCALE = 1.0 / math.sqrt(_DIM)
_LANES = 16
_NBUF = 2


def _embed_kernel(ids_hbm, table_hbm, out_hbm,
                  idx_v, gbufs, sbufs, gsems, ssems):
    wid = lax.axis_index("s") * 2 + lax.axis_index("c")
    base = wid * _PER_W

    pltpu.sync_copy(ids_hbm.at[wid], idx_v)

    def start_gather(j, b):
        pltpu.async_copy(table_hbm.at[idx_v.at[j]], gbufs[b], gsems[b])

    def scale(b):
        def row(i, _):
            for c in range(_DIM // _LANES):
                sl = pl.ds(c * _LANES, _LANES)
                sbufs[b][i, sl] = gbufs[b][i, sl] * _SCALE
            return 0
        lax.fori_loop(0, _C, row, 0, unroll=8)

    def step(j, b, wait_store, more_gathers):
        pltpu.make_async_copy(table_hbm.at[idx_v.at[j]], gbufs[b],
                              gsems[b]).wait()
        if wait_store:
            pltpu.make_async_copy(
                sbufs[b], out_hbm.at[pl.ds(base, _C)], ssems[b]).wait()
        scale(b)
        if more_gathers:
            start_gather(j + _NBUF, b)
        pltpu.async_copy(sbufs[b], out_hbm.at[pl.ds(base + j * _C, _C)],
                         ssems[b])

    for b in range(_NBUF):
        start_gather(b, b)
    for b in range(_NBUF):
        step(b, b, wait_store=False, more_gathers=True)

    def loop_body(jj, _):
        j = jj * _NBUF
        for b in range(_NBUF):
            step(j + b, b, wait_store=True, more_gathers=True)
        return 0
    lax.fori_loop(1, _NCHUNK // _NBUF - 1, loop_body, 0)

    for b in range(_NBUF):
        step(_NCHUNK - _NBUF + b, b, wait_store=True, more_gathers=False)
    for b in range(_NBUF):
        pltpu.make_async_copy(
            sbufs[b], out_hbm.at[pl.ds(base, _C)], ssems[b]).wait()


@jax.jit
def _embed(ids_grouped, W):
    mesh = plsc.VectorSubcoreMesh(core_axis_name="c", subcore_axis_name="s")
    flat = pl.kernel(
        _embed_kernel,
        mesh=mesh,
        out_type=jax.ShapeDtypeStruct((_N, _DIM), jnp.float32),
        scratch_types=[
            pltpu.VMEM((_NCHUNK, _C), jnp.int32),
            [pltpu.VMEM((_C, _DIM), jnp.float32) for _ in range(_NBUF)],
            [pltpu.VMEM((_C, _DIM), jnp.float32) for _ in range(_NBUF)],
            [pltpu.SemaphoreType.DMA for _ in range(_NBUF)],
            [pltpu.SemaphoreType.DMA for _ in range(_NBUF)],
        ],
        compiler_params=pltpu.CompilerParams(use_tc_tiling_on_sc=False),
    )(ids_grouped, W)
    return flat


def kernel(ids, W):
    ids_grouped = ids.astype(jnp.int32).reshape(_NW, _NCHUNK, _C)
    flat = _embed(ids_grouped, W)
    return flat.reshape(_B, _S, _DIM)


# speedup vs baseline: 1.4887x; 1.2632x over previous
import math

import jax
import jax.numpy as jnp
from jax import lax
from jax.experimental import pallas as pl
from jax.experimental.pallas import tpu as pltpu
from jax.experimental.pallas import tpu_sc as plsc

_VOCAB = 1000000
_DIM = 64
_B = 4096
_S = 200
_NW = 32
_ROWS_W = _B // _NW
_SCALE = 1.0 / math.sqrt(_DIM)
_LANES = 16
_NBUF = 2


def _embed_kernel(ids_hbm, table_hbm, out_hbm,
                  idx_v, gbufs, sbufs, gsems, ssems):
    wid = lax.axis_index("s") * 2 + lax.axis_index("c")
    row0 = wid * _ROWS_W

    pltpu.sync_copy(ids_hbm.at[pl.ds(row0, _ROWS_W)], idx_v)

    def start_gather(r, b):
        pltpu.async_copy(table_hbm.at[idx_v.at[r]], gbufs[b], gsems[b])

    def scale(b):
        @plsc.parallel_loop(0, _S, unroll=8)
        def _(i):
            for c in range(_DIM // _LANES):
                sl = pl.ds(c * _LANES, _LANES)
                sbufs[b][i, sl] = gbufs[b][i, sl] * _SCALE

    def step(r, b, wait_store, more_gathers):
        pltpu.make_async_copy(table_hbm.at[idx_v.at[r]], gbufs[b],
                              gsems[b]).wait()
        if wait_store:
            pltpu.make_async_copy(sbufs[b], out_hbm.at[row0], ssems[b]).wait()
        scale(b)
        if more_gathers:
            start_gather(r + _NBUF, b)
        pltpu.async_copy(sbufs[b], out_hbm.at[row0 + r], ssems[b])

    for b in range(_NBUF):
        start_gather(b, b)
    for b in range(_NBUF):
        step(b, b, wait_store=False, more_gathers=True)

    def loop_body(rr, _):
        r = rr * _NBUF
        for b in range(_NBUF):
            step(r + b, b, wait_store=True, more_gathers=True)
        return 0
    lax.fori_loop(1, _ROWS_W // _NBUF - 1, loop_body, 0)

    for b in range(_NBUF):
        step(_ROWS_W - _NBUF + b, b, wait_store=True, more_gathers=False)
    for b in range(_NBUF):
        pltpu.make_async_copy(sbufs[b], out_hbm.at[row0], ssems[b]).wait()


@jax.jit
def _embed(ids, W):
    mesh = plsc.VectorSubcoreMesh(core_axis_name="c", subcore_axis_name="s")
    return pl.kernel(
        _embed_kernel,
        mesh=mesh,
        out_type=jax.ShapeDtypeStruct((_B, _S, _DIM), jnp.float32),
        scratch_types=[
            pltpu.VMEM((_ROWS_W, _S), jnp.int32),
            [pltpu.VMEM((_S, _DIM), jnp.float32) for _ in range(_NBUF)],
            [pltpu.VMEM((_S, _DIM), jnp.float32) for _ in range(_NBUF)],
            [pltpu.SemaphoreType.DMA for _ in range(_NBUF)],
            [pltpu.SemaphoreType.DMA for _ in range(_NBUF)],
        ],
        compiler_params=pltpu.CompilerParams(use_tc_tiling_on_sc=False),
    )(ids, W)


def kernel(ids, W):
    return _embed(ids.astype(jnp.int32), W)


# speedup vs baseline: 2.3719x; 1.5933x over previous
import math

import jax
import jax.numpy as jnp
from jax import lax
from jax.experimental import pallas as pl
from jax.experimental.pallas import tpu as pltpu
from jax.experimental.pallas import tpu_sc as plsc

_VOCAB = 1000000
_DIM = 64
_B = 4096
_S = 200
_NW = 32
_BW = _B // _NW
_TB = _B // 128
_SCALE = 1.0 / math.sqrt(_DIM)
_L = 16
_NBUF = 2
_PITCH = 129


def _embed_kernel(idst_hbm, table_hbm, out_hbm,
                  idx_v, gbufs, obufs, gsems, ssems):
    wid = lax.axis_index("s") * 2 + lax.axis_index("c")

    pltpu.sync_copy(idst_hbm.at[:, pl.ds(wid * _BW, _BW)], idx_v)

    iota = lax.iota(jnp.int32, _L)
    dvecs = [iota + c * _L for c in range(_DIM // _L)]

    def start_gather(s, b):
        pltpu.async_copy(table_hbm.at[idx_v.at[s]], gbufs[b], gsems[b])

    def transpose_scale(b):
        gbuf, obuf = gbufs[b], obufs[b]

        @plsc.parallel_loop(0, _BW, unroll=4)
        def _(bm):
            bmv = iota * 0 + bm
            for c in range(_DIM // _L):
                v = gbuf[bm, pl.ds(c * _L, _L)] * _SCALE
                plsc.store_scatter(obuf, [dvecs[c], bmv], v)

    def store_out(s, b):
        for td in range(8):
            pltpu.async_copy(
                obufs[b].at[pl.ds(td * 8, 8), pl.ds(0, 128)],
                out_hbm.at[s, td, wid], ssems[b])

    def wait_stores(s, b):
        for td in range(8):
            pltpu.make_async_copy(
                obufs[b].at[pl.ds(td * 8, 8), pl.ds(0, 128)],
                out_hbm.at[s, td, wid], ssems[b]).wait()

    def step(s, b, wait_store, more_gathers):
        pltpu.make_async_copy(table_hbm.at[idx_v.at[s]], gbufs[b],
                              gsems[b]).wait()
        if wait_store:
            wait_stores(s, b)
        transpose_scale(b)
        if more_gathers:
            start_gather(s + _NBUF, b)
        store_out(s, b)

    for b in range(_NBUF):
        start_gather(b, b)
    for b in range(_NBUF):
        step(b, b, wait_store=False, more_gathers=True)

    def loop_body(ss, _):
        s = ss * _NBUF
        for b in range(_NBUF):
            step(s + b, b, wait_store=True, more_gathers=True)
        return 0
    lax.fori_loop(1, _S // _NBUF - 1, loop_body, 0)

    for b in range(_NBUF):
        step(_S - _NBUF + b, b, wait_store=True, more_gathers=False)
    for b in range(_NBUF):
        wait_stores(0, b)


@jax.jit
def _embed(ids_t, W):
    mesh = plsc.VectorSubcoreMesh(core_axis_name="c", subcore_axis_name="s")
    return pl.kernel(
        _embed_kernel,
        mesh=mesh,
        out_type=jax.ShapeDtypeStruct((_S, 8, _TB, 8, 128), jnp.float32),
        scratch_types=[
            pltpu.VMEM((_S, _BW), jnp.int32),
            [pltpu.VMEM((_BW, _DIM), jnp.float32) for _ in range(_NBUF)],
            [pltpu.VMEM((_DIM, _PITCH), jnp.float32) for _ in range(_NBUF)],
            [pltpu.SemaphoreType.DMA for _ in range(_NBUF)],
            [pltpu.SemaphoreType.DMA for _ in range(_NBUF)],
        ],
        compiler_params=pltpu.CompilerParams(use_tc_tiling_on_sc=False,
                                             needs_layout_passes=False),
    )(ids_t, W)


def kernel(ids, W):
    o5 = _embed(ids.astype(jnp.int32).T, W)
    return o5.transpose(2, 4, 0, 1, 3).reshape(_B, _S, _DIM)
